# baseline (speedup 1.0000x reference)
import jax
import jax.numpy as jnp
from jax import lax
from jax.experimental import pallas as pl
from jax.experimental.pallas import tpu as pltpu
from jax.experimental.pallas import tpu_sc as plsc

B = 4
V = 100000
F = 200000
F_PAD = 204800
C = 1600
L = 16
GROUPS = C // L
NC = 2
NS = 16
TASKS = 6
W = TASKS * F_PAD // NS
BLOCKS_PER_TILE = F_PAD // C // NS
VALID_BLOCKS = F // C
NBUF = 4


def _body(vn_hbm, faces_hbm, out_hbm, part_hbm,
          table_v, fb_v, ob_v, sem_in, sem_out):
    cid = lax.axis_index("c")
    sid = lax.axis_index("s")
    i3 = lax.iota(jnp.int32, L) * 3

    start = sid * W
    end = start + W
    task_a = start // F_PAD
    boundary = (task_a + 1) * F_PAD
    end_a = jnp.minimum(end, boundary)
    n_chunks_a = (end_a - start) // C
    n_chunks_b = (end - end_a) // C
    task_b = jnp.minimum(task_a + 1, TASKS - 1)

    def run_segment(task, face0, n_chunks):
        tbl_off = pl.multiple_of((cid * TASKS + task) * V, V)
        pltpu.sync_copy(vn_hbm.at[pl.ds(tbl_off, V)], table_v)
        pbase = (cid * TASKS + task) * F_PAD + face0

        def in_slices(k, sel):
            src_off = pl.multiple_of((face0 + k * C) * 3, 3 * C)
            dst_off = pl.multiple_of(sel * 3 * C, 3 * C)
            return (faces_hbm.at[pl.ds(src_off, 3 * C)],
                    fb_v.at[pl.ds(dst_off, 3 * C)])

        def out_slices(k, sel):
            dst_off = pl.multiple_of(pbase + k * C, C)
            return (ob_v.at[pl.ds(pl.multiple_of(sel * C, C), C)],
                    part_hbm.at[pl.ds(dst_off, C)])

        for d in range(NBUF - 1):
            @pl.when(d < n_chunks)
            def _():
                s, t = in_slices(d, d)
                pltpu.async_copy(s, t, sem_in)

        def chunk_body(k, _):
            sel = lax.rem(k, NBUF)
            s, t = in_slices(k, sel)
            pltpu.make_async_copy(s, t, sem_in).wait()

            kp = k + NBUF - 1
            @pl.when(kp < n_chunks)
            def _():
                s2, t2 = in_slices(kp, lax.rem(kp, NBUF))
                pltpu.async_copy(s2, t2, sem_in)

            @pl.when(k >= NBUF)
            def _():
                s3, t3 = out_slices(k - NBUF, sel)
                pltpu.make_async_copy(s3, t3, sem_out).wait()

            fb_base = sel * 3 * C
            ob_base = sel * C

            def grp(g, _):
                o = pl.multiple_of(g * L, L)
                gi = i3 + (fb_base + o * 3)
                a0 = plsc.bitcast(plsc.load_gather(fb_v, [gi]), jnp.int32)
                a1 = plsc.bitcast(plsc.load_gather(fb_v, [gi + 1]), jnp.int32)
                a2 = plsc.bitcast(plsc.load_gather(fb_v, [gi + 2]), jnp.int32)
                v0 = plsc.load_gather(table_v, [a0])
                v1 = plsc.load_gather(table_v, [a1])
                v2 = plsc.load_gather(table_v, [a2])
                s4 = v0 + v1 + v2
                ob_v[pl.ds(ob_base + o, L)] = (
                    s4 * s4 - v0 * v0 - v1 * v1 - v2 * v2)
                return 0

            lax.fori_loop(0, GROUPS, grp, 0, unroll=4)
            s5, t5 = out_slices(k, sel)
            pltpu.async_copy(s5, t5, sem_out)
            return 0

        lax.fori_loop(0, n_chunks, chunk_body, 0)

        for d in range(NBUF):
            j = n_chunks - NBUF + d

            @pl.when(j >= 0)
            def _():
                s6, t6 = out_slices(j, lax.rem(j, NBUF))
                pltpu.make_async_copy(s6, t6, sem_out).wait()

    run_segment(task_a, start - task_a * F_PAD, n_chunks_a)

    @pl.when(n_chunks_b > 0)
    def _():
        run_segment(task_b, 0, n_chunks_b)

    plsc.subcore_barrier()

    def unit_info(u):
        blk = sid * BLOCKS_PER_TILE + u // 2
        bat = u % 2
        off = pl.multiple_of(blk * C, C)
        base = pl.multiple_of((cid * TASKS + bat * 3) * F_PAD + off, C)
        valid = blk < VALID_BLOCKS
        return bat, off, base, valid

    def p2_in(u):
        _, _, base, _ = unit_info(u)
        sel2 = (u & 1) * 3 * C
        copies = []
        for r in range(3):
            copies.append((
                part_hbm.at[pl.ds(pl.multiple_of(base + r * F_PAD, C), C)],
                fb_v.at[pl.ds(pl.multiple_of(sel2 + r * C, C), C)],
            ))
        return copies

    def p2_out(u):
        bat, off, _, _ = unit_info(u)
        sel2 = (u & 1) * C
        out_off = pl.multiple_of((cid * 2 + bat) * F + off, C)
        return (ob_v.at[pl.ds(pl.multiple_of(sel2, C), C)],
                out_hbm.at[pl.ds(out_off, C)])

    NUNITS = 2 * BLOCKS_PER_TILE

    @pl.when(unit_info(0)[3])
    def _():
        for s, t in p2_in(0):
            pltpu.async_copy(s, t, sem_in)

    for u in range(NUNITS):
        bat, off, base, valid = unit_info(u)

        if u + 1 < NUNITS:
            @pl.when(unit_info(u + 1)[3])
            def _():
                for s, t in p2_in(u + 1):
                    pltpu.async_copy(s, t, sem_in)

        if u >= 2:
            @pl.when(unit_info(u - 2)[3])
            def _():
                s, t = p2_out(u - 2)
                pltpu.make_async_copy(s, t, sem_out).wait()

        @pl.when(valid)
        def _():
            for s, t in p2_in(u):
                pltpu.make_async_copy(s, t, sem_in).wait()
            fb_base = (u & 1) * 3 * C
            ob_base = (u & 1) * C

            def g2(g, _):
                o = pl.multiple_of(g * L, L)
                p0 = fb_v[pl.ds(fb_base + o, L)]
                p1 = fb_v[pl.ds(fb_base + C + o, L)]
                p2 = fb_v[pl.ds(fb_base + 2 * C + o, L)]
                ob_v[pl.ds(ob_base + o, L)] = 3.0 - 0.5 * (p0 + p1 + p2)
                return 0

            lax.fori_loop(0, GROUPS, g2, 0, unroll=4)
            s7, t7 = p2_out(u)
            pltpu.async_copy(s7, t7, sem_out)

    for u in (NUNITS - 2, NUNITS - 1):
        @pl.when(unit_info(u)[3])
        def _():
            s, t = p2_out(u)
            pltpu.make_async_copy(s, t, sem_out).wait()


@jax.jit
def kernel(vertex_normals, faces):
    faces = jnp.squeeze(faces)
    vn_flat = jnp.transpose(vertex_normals, (0, 2, 1)).reshape(B * 3 * V)
    faces_flat = lax.bitcast_convert_type(
        jnp.pad(faces, ((0, F_PAD - F), (0, 0))).reshape(3 * F_PAD),
        jnp.float32)

    mesh = plsc.VectorSubcoreMesh(
        core_axis_name="c", subcore_axis_name="s", num_cores=NC, num_subcores=NS
    )
    run = pl.kernel(
        _body,
        out_type=(
            jax.ShapeDtypeStruct((B * F,), jnp.float32),
            jax.ShapeDtypeStruct((B * 3 * F_PAD,), jnp.float32),
        ),
        mesh=mesh,
        compiler_params=pltpu.CompilerParams(needs_layout_passes=False),
        scratch_types=[
            pltpu.VMEM((V,), jnp.float32),
            pltpu.VMEM((NBUF * 3 * C,), jnp.float32),
            pltpu.VMEM((NBUF * C,), jnp.float32),
            pltpu.SemaphoreType.DMA,
            pltpu.SemaphoreType.DMA,
        ],
    )
    out, _ = run(vn_flat, faces_flat)
    return out.reshape(B, F)


# speedup vs baseline: 1.3400x; 1.3400x over previous
import jax
import jax.numpy as jnp
from jax import lax
from jax.experimental import pallas as pl
from jax.experimental.pallas import tpu as pltpu
from jax.experimental.pallas import tpu_sc as plsc

B = 4
V = 100000
F = 200000
F_PAD = 204800
C = 1600
L = 16
GROUPS = C // L
NC = 2
NS = 16
TASKS = 6
W = TASKS * F_PAD // NS
BLOCKS_PER_TILE = F_PAD // C // NS
VALID_BLOCKS = F // C
NBUF = 4


def _body(vn_hbm, faces_hbm, out_hbm, part_hbm,
          table_v, fb_v, ob_v, sem_in, sem_out):
    cid = lax.axis_index("c")
    sid = lax.axis_index("s")
    i3 = lax.iota(jnp.int32, L) * 3

    start = sid * W
    end = start + W
    task_a = start // F_PAD
    boundary = (task_a + 1) * F_PAD
    end_a = jnp.minimum(end, boundary)
    n_chunks_a = (end_a - start) // C
    n_chunks_b = (end - end_a) // C
    task_b = jnp.minimum(task_a + 1, TASKS - 1)

    def run_segment(task, face0, n_chunks):
        tbl_off = pl.multiple_of((cid * TASKS + task) * V, V)
        pltpu.sync_copy(vn_hbm.at[pl.ds(tbl_off, V)], table_v)
        pbase = (cid * TASKS + task) * F_PAD + face0

        def in_slices(k, sel):
            src_off = pl.multiple_of((face0 + k * C) * 3, 3 * C)
            dst_off = pl.multiple_of(sel * 3 * C, 3 * C)
            return (faces_hbm.at[pl.ds(src_off, 3 * C)],
                    fb_v.at[pl.ds(dst_off, 3 * C)])

        def out_slices(k, sel):
            dst_off = pl.multiple_of(pbase + k * C, C)
            return (ob_v.at[pl.ds(pl.multiple_of(sel * C, C), C)],
                    part_hbm.at[pl.ds(dst_off, C)])

        for d in range(NBUF - 1):
            @pl.when(d < n_chunks)
            def _():
                s, t = in_slices(d, d)
                pltpu.async_copy(s, t, sem_in)

        def chunk_body(k, _):
            sel = lax.rem(k, NBUF)
            s, t = in_slices(k, sel)
            pltpu.make_async_copy(s, t, sem_in).wait()

            kp = k + NBUF - 1
            @pl.when(kp < n_chunks)
            def _():
                s2, t2 = in_slices(kp, lax.rem(kp, NBUF))
                pltpu.async_copy(s2, t2, sem_in)

            @pl.when(k >= NBUF)
            def _():
                s3, t3 = out_slices(k - NBUF, sel)
                pltpu.make_async_copy(s3, t3, sem_out).wait()

            fb_base = sel * 3 * C
            ob_base = sel * C

            @plsc.parallel_loop(0, C, step=L, unroll=8)
            def _(o):
                o = pl.multiple_of(o, L)
                gi = i3 + (fb_base + o * 3)
                a0 = plsc.bitcast(plsc.load_gather(fb_v, [gi]), jnp.int32)
                a1 = plsc.bitcast(plsc.load_gather(fb_v, [gi + 1]), jnp.int32)
                a2 = plsc.bitcast(plsc.load_gather(fb_v, [gi + 2]), jnp.int32)
                v0 = plsc.load_gather(table_v, [a0])
                v1 = plsc.load_gather(table_v, [a1])
                v2 = plsc.load_gather(table_v, [a2])
                s4 = v0 + v1 + v2
                ob_v[pl.ds(ob_base + o, L)] = (
                    s4 * s4 - v0 * v0 - v1 * v1 - v2 * v2)
            s5, t5 = out_slices(k, sel)
            pltpu.async_copy(s5, t5, sem_out)
            return 0

        lax.fori_loop(0, n_chunks, chunk_body, 0)

        for d in range(NBUF):
            j = n_chunks - NBUF + d

            @pl.when(j >= 0)
            def _():
                s6, t6 = out_slices(j, lax.rem(j, NBUF))
                pltpu.make_async_copy(s6, t6, sem_out).wait()

    run_segment(task_a, start - task_a * F_PAD, n_chunks_a)

    @pl.when(n_chunks_b > 0)
    def _():
        run_segment(task_b, 0, n_chunks_b)

    plsc.subcore_barrier()

    def unit_info(u):
        blk = sid * BLOCKS_PER_TILE + u // 2
        bat = u % 2
        off = pl.multiple_of(blk * C, C)
        base = pl.multiple_of((cid * TASKS + bat * 3) * F_PAD + off, C)
        valid = blk < VALID_BLOCKS
        return bat, off, base, valid

    def p2_in(u):
        _, _, base, _ = unit_info(u)
        sel2 = (u & 1) * 3 * C
        copies = []
        for r in range(3):
            copies.append((
                part_hbm.at[pl.ds(pl.multiple_of(base + r * F_PAD, C), C)],
                fb_v.at[pl.ds(pl.multiple_of(sel2 + r * C, C), C)],
            ))
        return copies

    def p2_out(u):
        bat, off, _, _ = unit_info(u)
        sel2 = (u & 1) * C
        out_off = pl.multiple_of((cid * 2 + bat) * F + off, C)
        return (ob_v.at[pl.ds(pl.multiple_of(sel2, C), C)],
                out_hbm.at[pl.ds(out_off, C)])

    NUNITS = 2 * BLOCKS_PER_TILE

    @pl.when(unit_info(0)[3])
    def _():
        for s, t in p2_in(0):
            pltpu.async_copy(s, t, sem_in)

    for u in range(NUNITS):
        bat, off, base, valid = unit_info(u)

        if u + 1 < NUNITS:
            @pl.when(unit_info(u + 1)[3])
            def _():
                for s, t in p2_in(u + 1):
                    pltpu.async_copy(s, t, sem_in)

        if u >= 2:
            @pl.when(unit_info(u - 2)[3])
            def _():
                s, t = p2_out(u - 2)
                pltpu.make_async_copy(s, t, sem_out).wait()

        @pl.when(valid)
        def _():
            for s, t in p2_in(u):
                pltpu.make_async_copy(s, t, sem_in).wait()
            fb_base = (u & 1) * 3 * C
            ob_base = (u & 1) * C

            @plsc.parallel_loop(0, C, step=L, unroll=8)
            def _(o):
                o = pl.multiple_of(o, L)
                p0 = fb_v[pl.ds(fb_base + o, L)]
                p1 = fb_v[pl.ds(fb_base + C + o, L)]
                p2 = fb_v[pl.ds(fb_base + 2 * C + o, L)]
                ob_v[pl.ds(ob_base + o, L)] = 3.0 - 0.5 * (p0 + p1 + p2)
            s7, t7 = p2_out(u)
            pltpu.async_copy(s7, t7, sem_out)

    for u in (NUNITS - 2, NUNITS - 1):
        @pl.when(unit_info(u)[3])
        def _():
            s, t = p2_out(u)
            pltpu.make_async_copy(s, t, sem_out).wait()


@jax.jit
def kernel(vertex_normals, faces):
    faces = jnp.squeeze(faces)
    vn_flat = jnp.transpose(vertex_normals, (0, 2, 1)).reshape(B * 3 * V)
    faces_flat = lax.bitcast_convert_type(
        jnp.pad(faces, ((0, F_PAD - F), (0, 0))).reshape(3 * F_PAD),
        jnp.float32)

    mesh = plsc.VectorSubcoreMesh(
        core_axis_name="c", subcore_axis_name="s", num_cores=NC, num_subcores=NS
    )
    run = pl.kernel(
        _body,
        out_type=(
            jax.ShapeDtypeStruct((B * F,), jnp.float32),
            jax.ShapeDtypeStruct((B * 3 * F_PAD,), jnp.float32),
        ),
        mesh=mesh,
        compiler_params=pltpu.CompilerParams(needs_layout_passes=False),
        scratch_types=[
            pltpu.VMEM((V,), jnp.float32),
            pltpu.VMEM((NBUF * 3 * C,), jnp.float32),
            pltpu.VMEM((NBUF * C,), jnp.float32),
            pltpu.SemaphoreType.DMA,
            pltpu.SemaphoreType.DMA,
        ],
    )
    out, _ = run(vn_flat, faces_flat)
    return out.reshape(B, F)


# speedup vs baseline: 2.9269x; 2.1842x over previous
import jax
import jax.numpy as jnp
from jax import lax
from jax.experimental import pallas as pl
from jax.experimental.pallas import tpu as pltpu
from jax.experimental.pallas import tpu_sc as plsc

B = 4
V = 100000
F = 200000
F_PAD = 204800
C = 1600
L = 16
GROUPS = C // L
NC = 2
NS = 16
TASKS = 6
W = TASKS * F_PAD // NS
BLOCKS_PER_TILE = F_PAD // C // NS
VALID_BLOCKS = F // C
NBUF = 4


def _body(vn_hbm, faces_hbm, out_hbm, part_hbm,
          table_v, fb_v, ob_v, sem_in, sem_out):
    cid = lax.axis_index("c")
    sid = lax.axis_index("s")

    start = sid * W
    end = start + W
    task_a = start // F_PAD
    boundary = (task_a + 1) * F_PAD
    end_a = jnp.minimum(end, boundary)
    n_chunks_a = (end_a - start) // C
    n_chunks_b = (end - end_a) // C
    task_b = jnp.minimum(task_a + 1, TASKS - 1)

    def run_segment(task, face0, n_chunks):
        tbl_off = pl.multiple_of((cid * TASKS + task) * V, V)
        pltpu.sync_copy(vn_hbm.at[pl.ds(tbl_off, V)], table_v)
        pbase = (cid * TASKS + task) * F_PAD + face0

        def in_slices(k, sel):
            f0 = face0 + k * C
            pairs = []
            for r in range(3):
                src_off = pl.multiple_of(f0 + r * F_PAD, C)
                dst_off = pl.multiple_of(sel * 3 * C + r * C, C)
                pairs.append((faces_hbm.at[pl.ds(src_off, C)],
                              fb_v.at[pl.ds(dst_off, C)]))
            return pairs

        def out_slices(k, sel):
            dst_off = pl.multiple_of(pbase + k * C, C)
            return (ob_v.at[pl.ds(pl.multiple_of(sel * C, C), C)],
                    part_hbm.at[pl.ds(dst_off, C)])

        for d in range(NBUF - 1):
            @pl.when(d < n_chunks)
            def _():
                for s, t in in_slices(d, d):
                    pltpu.async_copy(s, t, sem_in)

        def chunk_body(k, _):
            sel = lax.rem(k, NBUF)
            for s, t in in_slices(k, sel):
                pltpu.make_async_copy(s, t, sem_in).wait()

            kp = k + NBUF - 1
            @pl.when(kp < n_chunks)
            def _():
                for s2, t2 in in_slices(kp, lax.rem(kp, NBUF)):
                    pltpu.async_copy(s2, t2, sem_in)

            @pl.when(k >= NBUF)
            def _():
                s3, t3 = out_slices(k - NBUF, sel)
                pltpu.make_async_copy(s3, t3, sem_out).wait()

            fb_base = sel * 3 * C
            ob_base = sel * C

            @plsc.parallel_loop(0, C, step=L, unroll=16)
            def _(o):
                o = pl.multiple_of(o, L)
                a0 = plsc.bitcast(fb_v[pl.ds(fb_base + o, L)], jnp.int32)
                a1 = plsc.bitcast(fb_v[pl.ds(fb_base + C + o, L)], jnp.int32)
                a2 = plsc.bitcast(fb_v[pl.ds(fb_base + 2 * C + o, L)], jnp.int32)
                v0 = plsc.load_gather(table_v, [a0])
                v1 = plsc.load_gather(table_v, [a1])
                v2 = plsc.load_gather(table_v, [a2])
                s4 = v0 + v1 + v2
                ob_v[pl.ds(ob_base + o, L)] = (
                    s4 * s4 - v0 * v0 - v1 * v1 - v2 * v2)
            s5, t5 = out_slices(k, sel)
            pltpu.async_copy(s5, t5, sem_out)
            return 0

        lax.fori_loop(0, n_chunks, chunk_body, 0)

        for d in range(NBUF):
            j = n_chunks - NBUF + d

            @pl.when(j >= 0)
            def _():
                s6, t6 = out_slices(j, lax.rem(j, NBUF))
                pltpu.make_async_copy(s6, t6, sem_out).wait()

    run_segment(task_a, start - task_a * F_PAD, n_chunks_a)

    @pl.when(n_chunks_b > 0)
    def _():
        run_segment(task_b, 0, n_chunks_b)

    plsc.subcore_barrier()

    def unit_info(u):
        blk = sid * BLOCKS_PER_TILE + u // 2
        bat = u % 2
        off = pl.multiple_of(blk * C, C)
        base = pl.multiple_of((cid * TASKS + bat * 3) * F_PAD + off, C)
        valid = blk < VALID_BLOCKS
        return bat, off, base, valid

    def p2_in(u):
        _, _, base, _ = unit_info(u)
        sel2 = (u & 1) * 3 * C
        copies = []
        for r in range(3):
            copies.append((
                part_hbm.at[pl.ds(pl.multiple_of(base + r * F_PAD, C), C)],
                fb_v.at[pl.ds(pl.multiple_of(sel2 + r * C, C), C)],
            ))
        return copies

    def p2_out(u):
        bat, off, _, _ = unit_info(u)
        sel2 = (u & 1) * C
        out_off = pl.multiple_of((cid * 2 + bat) * F + off, C)
        return (ob_v.at[pl.ds(pl.multiple_of(sel2, C), C)],
                out_hbm.at[pl.ds(out_off, C)])

    NUNITS = 2 * BLOCKS_PER_TILE

    @pl.when(unit_info(0)[3])
    def _():
        for s, t in p2_in(0):
            pltpu.async_copy(s, t, sem_in)

    for u in range(NUNITS):
        bat, off, base, valid = unit_info(u)

        if u + 1 < NUNITS:
            @pl.when(unit_info(u + 1)[3])
            def _():
                for s, t in p2_in(u + 1):
                    pltpu.async_copy(s, t, sem_in)

        if u >= 2:
            @pl.when(unit_info(u - 2)[3])
            def _():
                s, t = p2_out(u - 2)
                pltpu.make_async_copy(s, t, sem_out).wait()

        @pl.when(valid)
        def _():
            for s, t in p2_in(u):
                pltpu.make_async_copy(s, t, sem_in).wait()
            fb_base = (u & 1) * 3 * C
            ob_base = (u & 1) * C

            @plsc.parallel_loop(0, C, step=L, unroll=8)
            def _(o):
                o = pl.multiple_of(o, L)
                p0 = fb_v[pl.ds(fb_base + o, L)]
                p1 = fb_v[pl.ds(fb_base + C + o, L)]
                p2 = fb_v[pl.ds(fb_base + 2 * C + o, L)]
                ob_v[pl.ds(ob_base + o, L)] = 3.0 - 0.5 * (p0 + p1 + p2)
            s7, t7 = p2_out(u)
            pltpu.async_copy(s7, t7, sem_out)

    for u in (NUNITS - 2, NUNITS - 1):
        @pl.when(unit_info(u)[3])
        def _():
            s, t = p2_out(u)
            pltpu.make_async_copy(s, t, sem_out).wait()


@jax.jit
def kernel(vertex_normals, faces):
    faces = jnp.squeeze(faces)
    vn_flat = jnp.transpose(vertex_normals, (0, 2, 1)).reshape(B * 3 * V)
    faces_flat = lax.bitcast_convert_type(
        jnp.pad(jnp.transpose(faces), ((0, 0), (0, F_PAD - F))).reshape(
            3 * F_PAD),
        jnp.float32)

    mesh = plsc.VectorSubcoreMesh(
        core_axis_name="c", subcore_axis_name="s", num_cores=NC, num_subcores=NS
    )
    run = pl.kernel(
        _body,
        out_type=(
            jax.ShapeDtypeStruct((B * F,), jnp.float32),
            jax.ShapeDtypeStruct((B * 3 * F_PAD,), jnp.float32),
        ),
        mesh=mesh,
        compiler_params=pltpu.CompilerParams(needs_layout_passes=False),
        scratch_types=[
            pltpu.VMEM((V,), jnp.float32),
            pltpu.VMEM((NBUF * 3 * C,), jnp.float32),
            pltpu.VMEM((NBUF * C,), jnp.float32),
            pltpu.SemaphoreType.DMA,
            pltpu.SemaphoreType.DMA,
        ],
    )
    out, _ = run(vn_flat, faces_flat)
    return out.reshape(B, F)
